# Initial kernel scaffold; baseline (speedup 1.0000x reference)
import functools

import jax
import jax.numpy as jnp
from jax import lax
from jax.experimental import pallas as pl
from jax.experimental.pallas import tpu as pltpu
from jax.experimental.pallas import tpu_sc as plsc

NUM_STEPS = 8
EMBED = 16
HID = 96
NCLS = 9
BPUZ = 256
N = BPUZ * 81
E = N * 20

RB = 1296
NRB = N // RB
EB = 1536
NEB = E // EB

NC = 2
NS = 16
NW = NC * NS
PER_W = E // NW
CH = 96
NCH = PER_W // CH
ZR = 216
RPS = N // NS

_f32 = jnp.float32


def _dot(a, b):
    return jnp.dot(a, b, preferred_element_type=_f32)


def _enc_body(cost_ref, row_ref, col_ref, d1_ref, r1_ref, c1_ref, b1_ref,
              w2_ref, b2_ref, w3_ref, b3_ref, w4_ref, b4_ref,
              wx_ref, w1a_ref, w1b_ref,
              x_ref, xw_ref, a_ref, bt_ref):
    c10 = cost_ref[...]
    r = row_ref[0, 0, :]
    c = col_ref[0, 0, :]
    iota9 = lax.broadcasted_iota(jnp.int32, (RB, NCLS), 1)
    oh_r = (r[:, None] == iota9).astype(_f32)
    oh_c = (c[:, None] == iota9).astype(_f32)
    h = _dot(c10, d1_ref[...]) + _dot(oh_r, r1_ref[...]) + _dot(oh_c, c1_ref[...])
    h = jnp.maximum(h + b1_ref[...], 0.0)
    h = jnp.maximum(_dot(h, w2_ref[...]) + b2_ref[...], 0.0)
    h = jnp.maximum(_dot(h, w3_ref[...]) + b3_ref[...], 0.0)
    x = _dot(h, w4_ref[...]) + b4_ref[...]
    x_ref[...] = jnp.concatenate([x, jnp.zeros((RB, 32), _f32)], axis=1)
    xw_ref[...] = _dot(x, wx_ref[...])
    a_ref[...] = _dot(x, w1a_ref[...])
    bt_ref[...] = _dot(x, w1b_ref[...])


def _encoder(cost10, row3, col3, d1, r1, c1, b1, w2, b2, w3, b3, w4, b4,
             wx_p, w1a, w1b):
    full = lambda i: (0, 0)
    data = lambda i: (i, 0)
    d3 = lambda i: (i, 0, 0)
    return pl.pallas_call(
        _enc_body,
        grid=(NRB,),
        in_specs=[
            pl.BlockSpec((RB, NCLS + 1), data),
            pl.BlockSpec((1, 1, RB), d3),
            pl.BlockSpec((1, 1, RB), d3),
            pl.BlockSpec((NCLS + 1, HID), full),
            pl.BlockSpec((NCLS, HID), full),
            pl.BlockSpec((NCLS, HID), full),
            pl.BlockSpec((1, HID), full),
            pl.BlockSpec((HID, HID), full),
            pl.BlockSpec((1, HID), full),
            pl.BlockSpec((HID, HID), full),
            pl.BlockSpec((1, HID), full),
            pl.BlockSpec((HID, HID), full),
            pl.BlockSpec((1, HID), full),
            pl.BlockSpec((HID, 512), full),
            pl.BlockSpec((HID, HID), full),
            pl.BlockSpec((HID, HID), full),
        ],
        out_specs=[
            pl.BlockSpec((RB, 128), data),
            pl.BlockSpec((RB, 512), data),
            pl.BlockSpec((RB, HID), data),
            pl.BlockSpec((RB, HID), data),
        ],
        out_shape=[
            jax.ShapeDtypeStruct((N, 128), _f32),
            jax.ShapeDtypeStruct((N, 512), _f32),
            jax.ShapeDtypeStruct((N, HID), _f32),
            jax.ShapeDtypeStruct((N, HID), _f32),
        ],
    )(cost10, row3, col3, d1, r1, c1, b1, w2, b2, w3, b3, w4, b4,
      wx_p, w1a, w1b)


def _sc_gather_body(a_hbm, b_hbm, src_hbm, dst_hbm, out_hbm,
                    idxs_v, idxd_v, ra_v, rb_v, sema, semb):
    wid = lax.axis_index("s") * NC + lax.axis_index("c")
    base0 = wid * PER_W

    def chunk(k, carry):
        base = base0 + k * CH
        pltpu.sync_copy(src_hbm.at[pl.ds(base, CH)], idxs_v)
        pltpu.sync_copy(dst_hbm.at[pl.ds(base, CH)], idxd_v)
        cpa = pltpu.async_copy(a_hbm.at[idxs_v], ra_v, sema)
        cpb = pltpu.async_copy(b_hbm.at[idxd_v], rb_v, semb)
        cpa.wait()
        cpb.wait()

        def row(rr, inner):
            for j in range(HID // 16):
                sl = pl.ds(j * 16, 16)
                ra_v[rr, sl] = ra_v[rr, sl] + rb_v[rr, sl]
            return inner

        lax.fori_loop(0, CH, row, 0)
        pltpu.sync_copy(ra_v, out_hbm.at[pl.ds(base, CH)])
        return carry

    lax.fori_loop(0, NCH, chunk, 0)


def _sc_gather(a, b, src, dst):
    mesh = plsc.VectorSubcoreMesh(core_axis_name="c", subcore_axis_name="s")
    return pl.kernel(
        _sc_gather_body,
        out_type=jax.ShapeDtypeStruct((E, HID), _f32),
        mesh=mesh,
        scratch_types=[
            pltpu.VMEM((CH,), jnp.int32),
            pltpu.VMEM((CH,), jnp.int32),
            pltpu.VMEM((CH, HID), _f32),
            pltpu.VMEM((CH, HID), _f32),
            pltpu.SemaphoreType.DMA,
            pltpu.SemaphoreType.DMA,
        ],
    )(a, b, src, dst)


def _mlp_body(t_ref, b1_ref, w2_ref, b2_ref, w3_ref, b3_ref, w4_ref, b4_ref,
              out_ref):
    h = jnp.maximum(t_ref[...] + b1_ref[...], 0.0)
    h = jnp.maximum(_dot(h, w2_ref[...]) + b2_ref[...], 0.0)
    h = jnp.maximum(_dot(h, w3_ref[...]) + b3_ref[...], 0.0)
    out_ref[...] = _dot(h, w4_ref[...]) + b4_ref[...]


def _edge_mlp(t, b1, w2, b2, w3, b3, w4, b4):
    full = lambda i: (0, 0)
    data = lambda i: (i, 0)
    return pl.pallas_call(
        _mlp_body,
        grid=(NEB,),
        in_specs=[
            pl.BlockSpec((EB, HID), data),
            pl.BlockSpec((1, HID), full),
            pl.BlockSpec((HID, HID), full),
            pl.BlockSpec((1, HID), full),
            pl.BlockSpec((HID, HID), full),
            pl.BlockSpec((1, HID), full),
            pl.BlockSpec((HID, HID), full),
            pl.BlockSpec((1, HID), full),
        ],
        out_specs=pl.BlockSpec((EB, HID), data),
        out_shape=jax.ShapeDtypeStruct((E, HID), _f32),
    )(t, b1, w2, b2, w3, b3, w4, b4)


def _sc_scatter_body(msg_hbm, dst_hbm, out_hbm,
                     idx_v, rows_v, zbuf_v, acc_sh):
    cid = lax.axis_index("c")
    sid = lax.axis_index("s")
    wid = sid * NC + cid

    def zrow(rr, carry):
        for j in range(HID // 16):
            zbuf_v[rr, pl.ds(j * 16, 16)] = jnp.zeros((16,), _f32)
        return carry

    lax.fori_loop(0, ZR, zrow, 0)
    for j in range(RPS // ZR):
        pltpu.sync_copy(zbuf_v, acc_sh.at[pl.ds(sid * RPS + j * ZR, ZR)])
    plsc.subcore_barrier()

    base0 = wid * PER_W

    def chunk(k, carry):
        base = base0 + k * CH
        pltpu.sync_copy(dst_hbm.at[pl.ds(base, CH)], idx_v)
        pltpu.sync_copy(msg_hbm.at[pl.ds(base, CH)], rows_v)
        pltpu.sync_copy(rows_v, acc_sh.at[idx_v], add=True)
        return carry

    lax.fori_loop(0, NCH, chunk, 0)
    plsc.subcore_barrier()
    pltpu.sync_copy(acc_sh.at[pl.ds(sid * RPS, RPS)],
                    out_hbm.at[cid, pl.ds(sid * RPS, RPS)])


def _sc_scatter(msg, dst):
    mesh = plsc.VectorSubcoreMesh(core_axis_name="c", subcore_axis_name="s")
    return pl.kernel(
        _sc_scatter_body,
        out_type=jax.ShapeDtypeStruct((NC, N, HID), _f32),
        mesh=mesh,
        scratch_types=[
            pltpu.VMEM((CH,), jnp.int32),
            pltpu.VMEM((CH, HID), _f32),
            pltpu.VMEM((ZR, HID), _f32),
            pltpu.VMEM_SHARED((N, HID), _f32),
        ],
    )(msg, dst)


def _lstm_body(xw_ref, m0_ref, m1_ref, h_ref, c_ref,
               wm_ref, wh_ref, w1a_ref, w1b_ref, ow_ref, ob_ref, lab_ref,
               h2_ref, c2_ref, a_ref, bt_ref, preds_ref, loss_ref):
    m = m0_ref[...] + m1_ref[...]
    h = h_ref[...]
    g = xw_ref[...] + _dot(m, wm_ref[...]) + _dot(h, wh_ref[...])
    i_g = g[:, 0:128]
    f_g = g[:, 128:256]
    g_g = g[:, 256:384]
    o_g = g[:, 384:512]
    c_new = (jax.nn.sigmoid(f_g) * c_ref[...]
             + jax.nn.sigmoid(i_g) * jnp.tanh(g_g))
    h_new = jax.nn.sigmoid(o_g) * jnp.tanh(c_new)
    h2_ref[...] = h_new
    c2_ref[...] = c_new
    h96 = h_new[:, 0:HID]
    a_ref[...] = _dot(h96, w1a_ref[...])
    bt_ref[...] = _dot(h96, w1b_ref[...])
    logits = _dot(h96, ow_ref[...]) + ob_ref[...]
    mx = jnp.max(logits, axis=-1, keepdims=True)
    iota10 = lax.broadcasted_iota(jnp.int32, (RB, NCLS + 1), 1)
    preds = jnp.min(jnp.where(logits == mx, iota10, NCLS + 1), axis=-1)
    preds_ref[0, 0, :] = preds.astype(jnp.int32)
    lab = lab_ref[0, 0, :]
    lse = jnp.log(jnp.sum(jnp.exp(logits - mx), axis=-1)) + mx[:, 0]
    picked = jnp.sum(jnp.where(iota10 == lab[:, None], logits, 0.0), axis=-1)
    part = jnp.sum(lse - picked)
    loss_ref[...] = part * jnp.ones((1, 1, 128), _f32)


def _lstm_step(xw_p, m2, h, c, wm_p, wh_p, w1a, w1b, ow, ob, lab3):
    full = lambda i: (0, 0)
    data = lambda i: (i, 0)
    d3 = lambda i: (i, 0, 0)
    return pl.pallas_call(
        _lstm_body,
        grid=(NRB,),
        in_specs=[
            pl.BlockSpec((RB, 512), data),
            pl.BlockSpec((RB, HID), data),
            pl.BlockSpec((RB, HID), data),
            pl.BlockSpec((RB, 128), data),
            pl.BlockSpec((RB, 128), data),
            pl.BlockSpec((HID, 512), full),
            pl.BlockSpec((128, 512), full),
            pl.BlockSpec((HID, HID), full),
            pl.BlockSpec((HID, HID), full),
            pl.BlockSpec((HID, NCLS + 1), full),
            pl.BlockSpec((1, NCLS + 1), full),
            pl.BlockSpec((1, 1, RB), d3),
        ],
        out_specs=[
            pl.BlockSpec((RB, 128), data),
            pl.BlockSpec((RB, 128), data),
            pl.BlockSpec((RB, HID), data),
            pl.BlockSpec((RB, HID), data),
            pl.BlockSpec((1, 1, RB), d3),
            pl.BlockSpec((1, 1, 128), d3),
        ],
        out_shape=[
            jax.ShapeDtypeStruct((N, 128), _f32),
            jax.ShapeDtypeStruct((N, 128), _f32),
            jax.ShapeDtypeStruct((N, HID), _f32),
            jax.ShapeDtypeStruct((N, HID), _f32),
            jax.ShapeDtypeStruct((NRB, 1, RB), jnp.int32),
            jax.ShapeDtypeStruct((NRB, 1, 128), _f32),
        ],
    )(xw_p, m2[0], m2[1], h, c, wm_p, wh_p, w1a, w1b, ow, ob, lab3)


def _pad_gates(m):
    k = m.shape[0]
    return jnp.pad(m.reshape(k, 4, HID), ((0, 0), (0, 0), (0, 128 - HID))
                   ).reshape(k, 512)


def kernel(cost, labels, row, col, edge_index, digit_embed, row_embed,
           col_embed, in_W1, in_b1, in_W2, in_b2, in_W3, in_b3, in_W4, in_b4,
           msg_W1, msg_b1, msg_W2, msg_b2, msg_W3, msg_b3, msg_W4, msg_b4,
           W_ih, W_hh, out_W, out_b):
    cost10 = cost.reshape(N, NCLS + 1)
    row3 = row.reshape(NRB, 1, RB)
    col3 = col.reshape(NRB, 1, RB)
    lab3 = labels.reshape(NRB, 1, RB)
    src = edge_index[0]
    dst = edge_index[1]

    d1 = digit_embed @ in_W1[:EMBED]
    r1 = row_embed @ in_W1[EMBED:2 * EMBED]
    c1 = col_embed @ in_W1[2 * EMBED:3 * EMBED]
    wx_p = _pad_gates(W_ih[:, :HID].T)
    wm_p = _pad_gates(W_ih[:, HID:].T)
    wh_p = jnp.pad(_pad_gates(W_hh.T), ((0, 128 - HID), (0, 0)))
    w1a = msg_W1[:HID]
    w1b = msg_W1[HID:]
    b2 = lambda v: v.reshape(1, -1)

    x_pad, xw_p, a, bt = _encoder(
        cost10, row3, col3, d1, r1, c1, b2(in_b1), in_W2, b2(in_b2),
        in_W3, b2(in_b3), in_W4, b2(in_b4), wx_p, w1a, w1b)

    h = x_pad
    c = jnp.zeros((N, 128), _f32)
    preds_list = []
    loss_parts = []
    for _ in range(NUM_STEPS):
        t = _sc_gather(a, bt, src, dst)
        msg = _edge_mlp(t, b2(msg_b1), msg_W2, b2(msg_b2), msg_W3,
                        b2(msg_b3), msg_W4, b2(msg_b4))
        m2 = _sc_scatter(msg, dst)
        h, c, a, bt, preds_t, loss_t = _lstm_step(
            xw_p, m2, h, c, wm_p, wh_p, w1a, w1b, out_W,
            b2(out_b), lab3)
        preds_list.append(preds_t.reshape(N))
        loss_parts.append(jnp.sum(loss_t[:, 0, 0]))

    preds = jnp.stack(preds_list, 0)
    loss = sum(loss_parts) / jnp.float32(NUM_STEPS * N)
    return preds, loss


# speedup vs baseline: 1.0752x; 1.0752x over previous
import jax
import jax.numpy as jnp
from jax import lax
from jax.experimental import pallas as pl

NUM_STEPS = 8
EMBED = 16
HID = 96
NCLS = 9
BPUZ = 256
N = BPUZ * 81
E = N * 20

RB = 1296
NRB = N // RB
EB = 1536
NEB = E // EB

_f32 = jnp.float32


def _enc_body(cost_ref, row_ref, col_ref, de_ref, re_ref, ce_ref,
              w1_ref, b1_ref, w2_ref, b2_ref, w3_ref, b3_ref, w4_ref, b4_ref,
              x_ref):
    c10 = cost_ref[...]
    r = row_ref[0, 0, :]
    c = col_ref[0, 0, :]
    iota9 = lax.broadcasted_iota(jnp.int32, (RB, NCLS), 1)
    oh_r = (r[:, None] == iota9).astype(_f32)
    oh_c = (c[:, None] == iota9).astype(_f32)
    digits = jnp.dot(c10, de_ref[...])
    rows = jnp.dot(oh_r, re_ref[...])
    cols = jnp.dot(oh_c, ce_ref[...])
    cat = jnp.concatenate([digits, rows, cols], axis=1)
    h = jnp.maximum(jnp.dot(cat, w1_ref[...]) + b1_ref[...], 0.0)
    h = jnp.maximum(jnp.dot(h, w2_ref[...]) + b2_ref[...], 0.0)
    h = jnp.maximum(jnp.dot(h, w3_ref[...]) + b3_ref[...], 0.0)
    x_ref[...] = jnp.dot(h, w4_ref[...]) + b4_ref[...]


def _encoder(cost10, row3, col3, de, re, ce, w1, b1, w2, b2, w3, b3, w4, b4):
    full = lambda i: (0, 0)
    data = lambda i: (i, 0)
    d3 = lambda i: (i, 0, 0)
    return pl.pallas_call(
        _enc_body,
        grid=(NRB,),
        in_specs=[
            pl.BlockSpec((RB, NCLS + 1), data),
            pl.BlockSpec((1, 1, RB), d3),
            pl.BlockSpec((1, 1, RB), d3),
            pl.BlockSpec((NCLS + 1, EMBED), full),
            pl.BlockSpec((NCLS, EMBED), full),
            pl.BlockSpec((NCLS, EMBED), full),
            pl.BlockSpec((3 * EMBED, HID), full),
            pl.BlockSpec((1, HID), full),
            pl.BlockSpec((HID, HID), full),
            pl.BlockSpec((1, HID), full),
            pl.BlockSpec((HID, HID), full),
            pl.BlockSpec((1, HID), full),
            pl.BlockSpec((HID, HID), full),
            pl.BlockSpec((1, HID), full),
        ],
        out_specs=pl.BlockSpec((RB, HID), data),
        out_shape=jax.ShapeDtypeStruct((N, HID), _f32),
    )(cost10, row3, col3, de, re, ce, w1, b1, w2, b2, w3, b3, w4, b4)


def _mlp_body(hs_ref, hd_ref, w1_ref, b1_ref, w2_ref, b2_ref, w3_ref, b3_ref,
              w4_ref, b4_ref, out_ref):
    cat = jnp.concatenate([hs_ref[...], hd_ref[...]], axis=1)
    h = jnp.maximum(jnp.dot(cat, w1_ref[...]) + b1_ref[...], 0.0)
    h = jnp.maximum(jnp.dot(h, w2_ref[...]) + b2_ref[...], 0.0)
    h = jnp.maximum(jnp.dot(h, w3_ref[...]) + b3_ref[...], 0.0)
    out_ref[...] = jnp.dot(h, w4_ref[...]) + b4_ref[...]


def _edge_mlp(hs, hd, w1, b1, w2, b2, w3, b3, w4, b4):
    full = lambda i: (0, 0)
    data = lambda i: (i, 0)
    return pl.pallas_call(
        _mlp_body,
        grid=(NEB,),
        in_specs=[
            pl.BlockSpec((EB, HID), data),
            pl.BlockSpec((EB, HID), data),
            pl.BlockSpec((2 * HID, HID), full),
            pl.BlockSpec((1, HID), full),
            pl.BlockSpec((HID, HID), full),
            pl.BlockSpec((1, HID), full),
            pl.BlockSpec((HID, HID), full),
            pl.BlockSpec((1, HID), full),
            pl.BlockSpec((HID, HID), full),
            pl.BlockSpec((1, HID), full),
        ],
        out_specs=pl.BlockSpec((EB, HID), data),
        out_shape=jax.ShapeDtypeStruct((E, HID), _f32),
    )(hs, hd, w1, b1, w2, b2, w3, b3, w4, b4)


def _lstm_body(x_ref, m_ref, h_ref, c_ref, wiht_ref, whht_ref,
               ow_ref, ob_ref, lab_ref,
               h2_ref, c2_ref, preds_ref, loss_ref):
    xm = jnp.concatenate([x_ref[...], m_ref[...]], axis=1)
    g = jnp.dot(xm, wiht_ref[...]) + jnp.dot(h_ref[...], whht_ref[...])
    i_g = g[:, 0:HID]
    f_g = g[:, HID:2 * HID]
    g_g = g[:, 2 * HID:3 * HID]
    o_g = g[:, 3 * HID:4 * HID]
    c_new = (jax.nn.sigmoid(f_g) * c_ref[...]
             + jax.nn.sigmoid(i_g) * jnp.tanh(g_g))
    h_new = jax.nn.sigmoid(o_g) * jnp.tanh(c_new)
    h2_ref[...] = h_new
    c2_ref[...] = c_new
    logits = jnp.dot(h_new, ow_ref[...]) + ob_ref[...]
    mx = jnp.max(logits, axis=-1, keepdims=True)
    iota10 = lax.broadcasted_iota(jnp.int32, (RB, NCLS + 1), 1)
    preds = jnp.min(jnp.where(logits == mx, iota10, NCLS + 1), axis=-1)
    preds_ref[0, 0, :] = preds.astype(jnp.int32)
    lab = lab_ref[0, 0, :]
    lse = jnp.log(jnp.sum(jnp.exp(logits - mx), axis=-1)) + mx[:, 0]
    picked = jnp.sum(jnp.where(iota10 == lab[:, None], logits, 0.0), axis=-1)
    part = jnp.sum(lse - picked)
    loss_ref[...] = part * jnp.ones((1, 1, 128), _f32)


def _lstm_step(x, m, h, c, wiht, whht, ow, ob, lab3):
    full = lambda i: (0, 0)
    data = lambda i: (i, 0)
    d3 = lambda i: (i, 0, 0)
    return pl.pallas_call(
        _lstm_body,
        grid=(NRB,),
        in_specs=[
            pl.BlockSpec((RB, HID), data),
            pl.BlockSpec((RB, HID), data),
            pl.BlockSpec((RB, HID), data),
            pl.BlockSpec((RB, HID), data),
            pl.BlockSpec((2 * HID, 4 * HID), full),
            pl.BlockSpec((HID, 4 * HID), full),
            pl.BlockSpec((HID, NCLS + 1), full),
            pl.BlockSpec((1, NCLS + 1), full),
            pl.BlockSpec((1, 1, RB), d3),
        ],
        out_specs=[
            pl.BlockSpec((RB, HID), data),
            pl.BlockSpec((RB, HID), data),
            pl.BlockSpec((1, 1, RB), d3),
            pl.BlockSpec((1, 1, 128), d3),
        ],
        out_shape=[
            jax.ShapeDtypeStruct((N, HID), _f32),
            jax.ShapeDtypeStruct((N, HID), _f32),
            jax.ShapeDtypeStruct((NRB, 1, RB), jnp.int32),
            jax.ShapeDtypeStruct((NRB, 1, 128), _f32),
        ],
    )(x, m, h, c, wiht, whht, ow, ob, lab3)


def kernel(cost, labels, row, col, edge_index, digit_embed, row_embed,
           col_embed, in_W1, in_b1, in_W2, in_b2, in_W3, in_b3, in_W4, in_b4,
           msg_W1, msg_b1, msg_W2, msg_b2, msg_W3, msg_b3, msg_W4, msg_b4,
           W_ih, W_hh, out_W, out_b):
    cost10 = cost.reshape(N, NCLS + 1)
    row3 = row.reshape(NRB, 1, RB)
    col3 = col.reshape(NRB, 1, RB)
    lab3 = labels.reshape(NRB, 1, RB)
    src = edge_index[0]
    dst = edge_index[1]
    wiht = W_ih.T
    whht = W_hh.T
    b2 = lambda v: v.reshape(1, -1)

    x = _encoder(cost10, row3, col3, digit_embed, row_embed, col_embed,
                 in_W1, b2(in_b1), in_W2, b2(in_b2), in_W3, b2(in_b3),
                 in_W4, b2(in_b4))

    h = x
    rnn_h = jnp.zeros((N, HID), _f32)
    c = jnp.zeros((N, HID), _f32)
    preds_list = []
    loss_parts = []
    for _ in range(NUM_STEPS):
        msg = _edge_mlp(h[src], h[dst], msg_W1, b2(msg_b1), msg_W2,
                        b2(msg_b2), msg_W3, b2(msg_b3), msg_W4, b2(msg_b4))
        m = jnp.zeros((N, HID), _f32).at[dst].add(msg)
        rnn_h, c, preds_t, loss_t = _lstm_step(
            x, m, rnn_h, c, wiht, whht, out_W, b2(out_b), lab3)
        h = rnn_h
        preds_list.append(preds_t.reshape(N))
        loss_parts.append(jnp.sum(loss_t[:, 0, 0]))

    preds = jnp.stack(preds_list, 0)
    loss = sum(loss_parts) / jnp.float32(NUM_STEPS * N)
    return preds, loss
